# baseline (speedup 1.0000x reference)
import functools

import jax
import jax.numpy as jnp
from jax import lax
from jax.experimental import pallas as pl
from jax.experimental.pallas import tpu as pltpu
from jax.experimental.pallas import tpu_sc as plsc

_L = 4096
_W = 2 * _L
_CW = _W + 16
_NW = 32
_RPW = _L // _NW
_THRESH = (19, 22, 25, 28, 32, 37, 43, 49, 56, 64, 74, 85, 98, 112, 128)

_mesh = plsc.VectorSubcoreMesh(core_axis_name="c", subcore_axis_name="s")


def _take16(vec, idx):
    return lax.gather(
        vec,
        idx[:, None],
        lax.GatherDimensionNumbers(
            offset_dims=(), collapsed_slice_dims=(0,), start_index_map=(0,)
        ),
        slice_sizes=(1,),
        mode=lax.GatherScatterMode.PROMISE_IN_BOUNDS,
    )


@functools.partial(
    pl.kernel,
    mesh=_mesh,
    out_type=jax.ShapeDtypeStruct((_L * _L,), jnp.float32),
    scratch_types=[
        pltpu.VMEM((32,), jnp.float32),
        pltpu.VMEM((_W,), jnp.float32),
    ]
    + [pltpu.VMEM((_CW,), jnp.float32) for _ in range(8)]
    + [pltpu.SemaphoreType.DMA],
)
def _sc_bias(w_hbm, out_hbm, w_v, v_v, *c_and_sem):
    c_v = c_and_sem[:8]
    sem = c_and_sem[8]
    pltpu.sync_copy(w_hbm, w_v)

    lanes = lax.iota(jnp.int32, 16)
    w0 = w_v[pl.ds(0, 16)]
    w1 = w_v[pl.ds(16, 16)]

    def compute_v(j, carry):
        t = lanes + j * 16
        rel = t - (_L - 1)
        dist = jnp.abs(rel)
        b = jnp.full((16,), 16, jnp.int32)
        for thr in _THRESH:
            b = b + jnp.where(dist >= thr, 1, 0)
        b = jnp.where(dist < 16, dist, b)
        b = b + jnp.where(rel < 0, 16, 0)
        b = jnp.minimum(b, 31)
        hi = b >= 16
        bl = jnp.where(hi, b - 16, b)
        v_v[pl.ds(j * 16, 16)] = jnp.where(
            hi, _take16(w1, bl), _take16(w0, bl)
        )
        return carry

    lax.fori_loop(0, _W // 16, compute_v, 0)

    val0 = v_v[pl.ds(0, 16)]
    for s in range(8):
        c_v[s][pl.ds(0, 16)] = _take16(val0, jnp.maximum(lanes - (s + 1), 0))

    def build_shifts(j, carry):
        t0 = j * 16
        for s in range(8):
            c_v[s][pl.ds(t0, 16)] = v_v[pl.ds(t0 - (s + 1), 16)]
        return carry

    lax.fori_loop(1, _W // 16, build_shifts, 0)

    wid = lax.axis_index("s") * 2 + lax.axis_index("c")
    i0 = wid * _RPW

    def emit_rows(j, carry):
        off = _L - i0 - j * 8
        copies = []
        for u in range(8):
            i = i0 + j * 8 + u
            copies.append(
                pltpu.make_async_copy(
                    c_v[u].at[pl.ds(off, _L)],
                    out_hbm.at[pl.ds(i * _L, _L)],
                    sem,
                )
            )
        for cp in copies:
            cp.start()
        for cp in copies:
            cp.wait()
        return carry

    lax.fori_loop(0, _RPW // 8, emit_rows, 0)


def kernel(weight, L):
    del L
    return _sc_bias(weight.reshape(32)).reshape(_L, _L, 1)


# speedup vs baseline: 1.1038x; 1.1038x over previous
import functools

import jax
import jax.numpy as jnp
from jax import lax
from jax.experimental import pallas as pl
from jax.experimental.pallas import tpu as pltpu
from jax.experimental.pallas import tpu_sc as plsc

_L = 4096
_W = 2 * _L
_NW = 32
_CPW = _W // 16 // _NW
_ROWS = 128
_GRID = _L // _ROWS
_SL = _W // 128
_THRESH = (19, 22, 25, 28, 32, 37, 43, 49, 56, 64, 74, 85, 98, 112, 128)

_mesh = plsc.VectorSubcoreMesh(core_axis_name="c", subcore_axis_name="s")


def _take16(vec, idx):
    return lax.gather(
        vec,
        idx[:, None],
        lax.GatherDimensionNumbers(
            offset_dims=(), collapsed_slice_dims=(0,), start_index_map=(0,)
        ),
        slice_sizes=(1,),
        mode=lax.GatherScatterMode.PROMISE_IN_BOUNDS,
    )


@functools.partial(
    pl.kernel,
    mesh=_mesh,
    out_type=jax.ShapeDtypeStruct((_W,), jnp.float32),
    scratch_types=[
        pltpu.VMEM((32,), jnp.float32),
        pltpu.VMEM((16 * _CPW,), jnp.float32),
        pltpu.SemaphoreType.DMA,
    ],
)
def _sc_lookup(w_hbm, out_hbm, w_v, v_v, sem):
    pltpu.sync_copy(w_hbm, w_v)

    lanes = lax.iota(jnp.int32, 16)
    w0 = w_v[pl.ds(0, 16)]
    w1 = w_v[pl.ds(16, 16)]
    wid = lax.axis_index("s") * 2 + lax.axis_index("c")
    t0 = wid * (16 * _CPW)

    def compute_v(j, carry):
        t = lanes + t0 + j * 16
        rel = t - (_L - 1)
        dist = jnp.abs(rel)
        b = jnp.full((16,), 16, jnp.int32)
        for thr in _THRESH:
            b = b + jnp.where(dist >= thr, 1, 0)
        b = jnp.where(dist < 16, dist, b)
        b = b + jnp.where(rel < 0, 16, 0)
        b = jnp.minimum(b, 31)
        hi = b >= 16
        bl = jnp.where(hi, b - 16, b)
        v_v[pl.ds(j * 16, 16)] = jnp.where(
            hi, _take16(w1, bl), _take16(w0, bl)
        )
        return carry

    lax.fori_loop(0, _CPW, compute_v, 0)
    cp = pltpu.make_async_copy(v_v, out_hbm.at[pl.ds(t0, 16 * _CPW)], sem)
    cp.start()
    cp.wait()


def _shift_flat(x, n):
    rolled = jnp.roll(x, n, axis=-1)
    down = jnp.concatenate(
        [jnp.zeros_like(rolled[..., :1, :]), rolled[..., :-1, :]], axis=-2
    )
    lane = jax.lax.broadcasted_iota(jnp.int32, x.shape, x.ndim - 1)
    return jnp.where(lane >= n, rolled, down)


def _expand_kernel(v_ref, out_ref, v2_ref, sem_ref):
    v2_ref[0:1] = _shift_flat(v_ref[...], 1)[None]
    n = 1
    while n < _ROWS:
        v2_ref[n : 2 * n] = _shift_flat(v2_ref[0:n], n)
        n *= 2

    copies = []
    for g in range(_GRID):
        copies.append(
            pltpu.make_async_copy(
                v2_ref.at[:, pl.ds(_GRID - g, _GRID), :],
                out_ref.at[pl.ds(g * _ROWS, _ROWS), :, :],
                sem_ref,
            )
        )
    for cp in copies:
        cp.start()
    for cp in copies:
        cp.wait()


@jax.jit
def _bias(weight):
    v = _sc_lookup(weight.reshape(32)).reshape(_SL, 128)
    return pl.pallas_call(
        _expand_kernel,
        grid=(1,),
        in_specs=[pl.BlockSpec(memory_space=pltpu.MemorySpace.VMEM)],
        out_specs=pl.BlockSpec(memory_space=pltpu.MemorySpace.HBM),
        out_shape=jax.ShapeDtypeStruct((_L, _GRID, 128), jnp.float32),
        scratch_shapes=[
            pltpu.VMEM((_ROWS, _SL, 128), jnp.float32),
            pltpu.SemaphoreType.DMA,
        ],
    )(v)


def kernel(weight, L):
    del L
    return _bias(weight).reshape(_L, _L, 1)


# speedup vs baseline: 1.1043x; 1.0004x over previous
import functools

import jax
import jax.numpy as jnp
from jax import lax
from jax.experimental import pallas as pl
from jax.experimental.pallas import tpu as pltpu
from jax.experimental.pallas import tpu_sc as plsc

_L = 4096
_W = 2 * _L
_NW = 32
_CPW = _W // 16 // _NW
_ROWS = 128
_GRID = _L // _ROWS
_SL = _W // 128
_THRESH = (19, 22, 25, 28, 32, 37, 43, 49, 56, 64, 74, 85, 98, 112, 128)

_mesh = plsc.VectorSubcoreMesh(core_axis_name="c", subcore_axis_name="s")


def _take16(vec, idx):
    return lax.gather(
        vec,
        idx[:, None],
        lax.GatherDimensionNumbers(
            offset_dims=(), collapsed_slice_dims=(0,), start_index_map=(0,)
        ),
        slice_sizes=(1,),
        mode=lax.GatherScatterMode.PROMISE_IN_BOUNDS,
    )


@functools.partial(
    pl.kernel,
    mesh=_mesh,
    out_type=jax.ShapeDtypeStruct((_W,), jnp.float32),
    scratch_types=[
        pltpu.VMEM((32,), jnp.float32),
        pltpu.VMEM((16 * _CPW,), jnp.float32),
        pltpu.SemaphoreType.DMA,
    ],
)
def _sc_lookup(w_hbm, out_hbm, w_v, v_v, sem):
    pltpu.sync_copy(w_hbm, w_v)

    lanes = lax.iota(jnp.int32, 16)
    w0 = w_v[pl.ds(0, 16)]
    w1 = w_v[pl.ds(16, 16)]
    wid = lax.axis_index("s") * 2 + lax.axis_index("c")
    t0 = wid * (16 * _CPW)

    def compute_v(j, carry):
        t = lanes + t0 + j * 16
        rel = t - (_L - 1)
        dist = jnp.abs(rel)
        b = jnp.full((16,), 16, jnp.int32)
        for thr in _THRESH:
            b = b + jnp.where(dist >= thr, 1, 0)
        b = jnp.where(dist < 16, dist, b)
        b = b + jnp.where(rel < 0, 16, 0)
        b = jnp.minimum(b, 31)
        hi = b >= 16
        bl = jnp.where(hi, b - 16, b)
        v_v[pl.ds(j * 16, 16)] = jnp.where(
            hi, _take16(w1, bl), _take16(w0, bl)
        )
        return carry

    lax.fori_loop(0, _CPW, compute_v, 0)
    cp = pltpu.make_async_copy(v_v, out_hbm.at[pl.ds(t0, 16 * _CPW)], sem)
    cp.start()
    cp.wait()


def _shift_flat(x, n):
    rolled = jnp.roll(x, n, axis=-1)
    down = jnp.concatenate(
        [jnp.zeros_like(rolled[..., :1, :]), rolled[..., :-1, :]], axis=-2
    )
    lane = jax.lax.broadcasted_iota(jnp.int32, x.shape, x.ndim - 1)
    return jnp.where(lane >= n, rolled, down)


def _expand_kernel(v_ref, out_ref, v2_ref, sem_ref):
    v2_ref[0:1] = _shift_flat(v_ref[...], 1)[None]
    n = 1
    while n < _ROWS:
        v2_ref[n : 2 * n] = _shift_flat(v2_ref[0:n], n)
        n *= 2

    copies = []
    for g2 in range(2 * _GRID):
        copies.append(
            pltpu.make_async_copy(
                v2_ref.at[
                    pl.ds(64 * (g2 % 2), 64), pl.ds(_GRID - g2 // 2, _GRID), :
                ],
                out_ref.at[pl.ds(g2 * 64, 64), :, :],
                sem_ref,
            )
        )
    for cp in copies:
        cp.start()
    for cp in copies:
        cp.wait()


@jax.jit
def _bias(weight):
    v = _sc_lookup(weight.reshape(32)).reshape(_SL, 128)
    return pl.pallas_call(
        _expand_kernel,
        grid=(1,),
        in_specs=[pl.BlockSpec(memory_space=pltpu.MemorySpace.VMEM)],
        out_specs=pl.BlockSpec(memory_space=pltpu.MemorySpace.HBM),
        out_shape=jax.ShapeDtypeStruct((_L, _GRID, 128), jnp.float32),
        scratch_shapes=[
            pltpu.VMEM((_ROWS, _SL, 128), jnp.float32),
            pltpu.SemaphoreType.DMA,
        ],
    )(v)


def kernel(weight, L):
    del L
    return _bias(weight).reshape(_L, _L, 1)
